# baseline (speedup 1.0000x reference)
import functools

import jax
import jax.numpy as jnp
from jax import lax
from jax.experimental import pallas as pl
from jax.experimental.pallas import tpu as pltpu
from jax.experimental.pallas import tpu_sc as plsc

N_ENT = 10000
N_USR = 4096
N_EDGE = 320000
CH = 128
DK = 64
N_REL11 = 11

NC = 2
NS = 16
NW = NC * NS
EPW = N_EDGE // NW
B = 80
NG = B // 16
NCHUNK = EPW // B
CROWS = 160
ACC_ROWS = N_ENT + CROWS


def _pq_body(x_ref, w_ref, o_ref):
    o_ref[...] = jnp.dot(x_ref[...], w_ref[...],
                         preferred_element_type=jnp.float32)


def _pq(x, w):
    bm = 2000
    return pl.pallas_call(
        _pq_body,
        grid=(N_ENT // bm,),
        in_specs=[
            pl.BlockSpec((bm, CH), lambda m: (m, 0)),
            pl.BlockSpec((CH, CH), lambda m: (0, 0)),
        ],
        out_specs=pl.BlockSpec((bm, CH), lambda m: (m, 0)),
        out_shape=jax.ShapeDtypeStruct((N_ENT, CH), jnp.float32),
    )(x, w)


def _user_body(a_ref, b_ref, o_ref):
    acc = jnp.dot(a_ref[...], b_ref[...], preferred_element_type=jnp.float32)
    n = jnp.sqrt(jnp.sum(acc * acc, axis=1, keepdims=True))
    o_ref[...] = acc / jnp.maximum(n, 1e-12)


def _user_mm(interact, ent):
    bm = 512
    return pl.pallas_call(
        _user_body,
        grid=(N_USR // bm,),
        in_specs=[
            pl.BlockSpec((bm, N_ENT), lambda m: (m, 0)),
            pl.BlockSpec((N_ENT, CH), lambda m: (0, 0)),
        ],
        out_specs=pl.BlockSpec((bm, CH), lambda m: (m, 0)),
        out_shape=jax.ShapeDtypeStruct((N_USR, CH), jnp.float32),
    )(interact, ent)


def _per_body(p_ref, e_ref, r_ref, o_ref):
    p = p_ref[...]
    en = e_ref[...]
    rl = r_ref[...]
    pr = (p[:, None, :] * rl[None, :, :]).reshape(-1, CH)
    er = (en[:, None, :] * rl[None, :, :]).reshape(-1, CH)
    o_ref[:, :CH] = pr.astype(jnp.bfloat16)
    o_ref[:, CH:] = er.astype(jnp.bfloat16)


def _per(p, ent, rel):
    bm = 1000
    return pl.pallas_call(
        _per_body,
        grid=(N_ENT // bm,),
        in_specs=[
            pl.BlockSpec((bm, CH), lambda m: (m, 0)),
            pl.BlockSpec((bm, CH), lambda m: (m, 0)),
            pl.BlockSpec((N_REL11, CH), lambda m: (0, 0)),
        ],
        out_specs=pl.BlockSpec((bm * N_REL11, 2 * CH), lambda m: (m, 0)),
        out_shape=jax.ShapeDtypeStruct((N_ENT * N_REL11, 2 * CH),
                                       jnp.bfloat16),
    )(p, ent, rel)


def _combine_body(a_ref, b_ref, s_ref, o_ref):
    u = a_ref[0] + b_ref[0]
    s0 = s_ref[:, 0:1]
    s1 = s_ref[:, 1:2]
    lanes = lax.broadcasted_iota(jnp.int32, u.shape, 1)
    denom = jnp.where(lanes < DK, s0, s1) + 1e-16
    agg = u / denom
    n = jnp.sqrt(jnp.sum(agg * agg, axis=1, keepdims=True))
    o_ref[...] = agg / jnp.maximum(n, 1e-12)


def _combine(eo, s):
    bm = 2000
    return pl.pallas_call(
        _combine_body,
        grid=(N_ENT // bm,),
        in_specs=[
            pl.BlockSpec((1, bm, CH), lambda m: (0, m, 0)),
            pl.BlockSpec((1, bm, CH), lambda m: (1, m, 0)),
            pl.BlockSpec((bm, 2), lambda m: (m, 0)),
        ],
        out_specs=pl.BlockSpec((bm, CH), lambda m: (m, 0)),
        out_shape=jax.ShapeDtypeStruct((N_ENT, CH), jnp.float32),
    )(eo, eo, s)


_MESH = plsc.VectorSubcoreMesh(core_axis_name="c", subcore_axis_name="s")


CNT_R = 160


@functools.partial(
    pl.kernel,
    out_type=pltpu.HBM((NC, ACC_ROWS, CH), jnp.float32),
    mesh=_MESH,
    compiler_params=pltpu.CompilerParams(needs_layout_passes=False),
    scratch_types=[
        pltpu.VMEM_SHARED((ACC_ROWS, CH), jnp.float32),
        pltpu.VMEM((B,), jnp.int32),
        pltpu.VMEM((B,), jnp.int32),
        pltpu.VMEM((B,), jnp.int32),
        pltpu.VMEM((B, CH), jnp.float32),
        pltpu.VMEM((B, CH), jnp.float32),
        pltpu.VMEM((B, CH), jnp.float32),
        pltpu.VMEM((B, CH), jnp.float32),
        pltpu.SemaphoreType.DMA,
        pltpu.SemaphoreType.DMA,
        pltpu.SemaphoreType.DMA,
    ],
)
def _edge_sc(ttab, pertab, eidx, zinit, out,
             acc, hv, tv, hv3, ph, tb, ov, erow, sm1, sm2, sm3):
    cid = lax.axis_index("c")
    sid = lax.axis_index("s")
    wid = sid * NC + cid

    lanes = lax.iota(jnp.int32, 16)
    m0 = lanes == 0
    mj = [lanes == j for j in range(16)]
    zv = jnp.zeros((16,), jnp.float32)

    @pl.when(sid == 0)
    def _():
        pltpu.sync_copy(zinit, acc)

    def zrow(r, carry):
        for c in range(8):
            erow[r, pl.ds(c * 16, 16)] = zv
        return carry

    lax.fori_loop(0, B, zrow, 0)
    plsc.subcore_barrier()

    def up2(v):
        return plsc.unpack(v, format=plsc.PackFormat.INTERLEAVED,
                           preferred_element_type=jnp.float32)

    def chunk(k, carry):
        c2i = (wid * NCHUNK + k) * 2
        pltpu.sync_copy(eidx.at[c2i], hv)
        pltpu.sync_copy(eidx.at[c2i + 1], tv)
        c1 = pltpu.async_copy(ttab.at[hv], ph, sm1)
        c2 = pltpu.async_copy(pertab.at[tv], tb, sm2)
        c1.wait()
        c2.wait()

        def group(g, gcarry):
            h16 = hv[pl.ds(g * 16, 16)]
            iv = lanes + g * 16
            e0v = zv
            e1v = zv
            for j in range(16):
                i = g * 16 + j
                kf = []
                for c in range(4):
                    pb = plsc.bitcast(ph[i, pl.ds(c * 16, 16)], jnp.bfloat16)
                    tbw = plsc.bitcast(tb[i, pl.ds(c * 16, 16)], jnp.bfloat16)
                    kf.extend(up2(pb * tbw))
                s0 = jnp.sum((kf[0] + kf[1]) + (kf[2] + kf[3]))
                s1 = jnp.sum((kf[4] + kf[5]) + (kf[6] + kf[7]))
                ev = jnp.exp(jnp.where(m0, s0, s1) * 0.125)
                e0 = ev[0]
                e1 = ev[1]
                for c in range(4):
                    vb = plsc.bitcast(tb[i, pl.ds(DK + c * 16, 16)],
                                      jnp.bfloat16)
                    lo, hi = up2(vb)
                    e = e0 if c < 2 else e1
                    ov[i, pl.ds(c * 32, 16)] = lo * e
                    ov[i, pl.ds(c * 32 + 16, 16)] = hi * e
                e0v = jnp.where(mj[j], e0, e0v)
                e1v = jnp.where(mj[j], e1, e1v)
            colv = (h16 & 63) * 2
            plsc.store_scatter(erow, [iv, colv], e0v)
            plsc.store_scatter(erow, [iv, colv + 1], e1v)
            return gcarry

        lax.fori_loop(0, NG, group, 0)

        def cntidx(g, gcarry):
            h16 = hv[pl.ds(g * 16, 16)]
            hv3[pl.ds(g * 16, 16)] = lax.shift_right_logical(h16, 6) + N_ENT
            return gcarry

        lax.fori_loop(0, NG, cntidx, 0)
        pltpu.sync_copy(ov, acc.at[hv], add=True)
        pltpu.sync_copy(erow, acc.at[hv3], add=True)

        def gz(g, gcarry):
            h16 = hv[pl.ds(g * 16, 16)]
            iv = lanes + g * 16
            colv = (h16 & 63) * 2
            plsc.store_scatter(erow, [iv, colv], zv)
            plsc.store_scatter(erow, [iv, colv + 1], zv)
            return gcarry

        lax.fori_loop(0, NG, gz, 0)
        return carry

    lax.fori_loop(0, NCHUNK, chunk, 0)
    plsc.subcore_barrier()

    @pl.when(sid == 0)
    def _():
        pltpu.sync_copy(acc, out.at[cid])


def _ileave(x):
    n = x.shape[0]
    y = x.reshape(n, CH // 32, 2, 16).transpose(0, 1, 3, 2).reshape(n, CH)
    y16 = y.astype(jnp.bfloat16).reshape(n, CH // 2, 2)
    return lax.bitcast_convert_type(y16, jnp.int32)


def kernel(user_emb, entity_emb, edge_index, edge_type, interact_mat,
           relation_emb, W_Q):
    head = edge_index[0]
    tail = edge_index[1]
    ridx = (edge_type + (N_REL11 - 1)) % N_REL11
    eidx = (jnp.stack([head, tail * N_REL11 + ridx])
            .reshape(2, NW, NCHUNK, B).transpose(1, 2, 0, 3)
            .reshape(NW * NCHUNK * 2, B))
    perm = jnp.array([32 * (q // 32) + 16 * (q % 2) + (q % 32) // 2
                      for q in range(CH)], jnp.int32)
    relp = relation_emb[:, perm]
    zinit = jnp.zeros((ACC_ROWS, CH), jnp.float32)

    ent = entity_emb
    ent_res = entity_emb
    usr_res = user_emb
    for _ in range(2):
        p = _pq(ent, W_Q)
        ttab = lax.bitcast_convert_type(
            jnp.concatenate([_ileave(p), _ileave(ent)], axis=1), jnp.float32)
        per = _per(p[:, perm], ent[:, perm], relp)
        pertab = lax.bitcast_convert_type(
            per.reshape(N_ENT * N_REL11, CH, 2), jnp.float32)
        eo = _edge_sc(ttab, pertab, eidx, zinit)
        s = (eo[0, N_ENT:] + eo[1, N_ENT:]).reshape(-1, 2)[:N_ENT]
        usr = _user_mm(interact_mat, ent)
        ent = _combine(eo, s)
        ent_res = ent_res + ent
        usr_res = usr_res + usr
    return ent_res, usr_res


# speedup vs baseline: 1.6247x; 1.6247x over previous
import functools

import jax
import jax.numpy as jnp
from jax import lax
from jax.experimental import pallas as pl
from jax.experimental.pallas import tpu as pltpu
from jax.experimental.pallas import tpu_sc as plsc

N_ENT = 10000
N_USR = 4096
N_EDGE = 320000
CH = 128
DK = 64
N_REL11 = 11

NC = 2
NS = 16
NW = NC * NS
EPW = N_EDGE // NW
B = 80
NG = B // 16
NCHUNK = EPW // B
CROWS = 160
ACC_ROWS = N_ENT + CROWS


def _pq_body(x_ref, w_ref, o_ref):
    o_ref[...] = jnp.dot(x_ref[...], w_ref[...],
                         preferred_element_type=jnp.float32)


def _pq(x, w):
    bm = 2000
    return pl.pallas_call(
        _pq_body,
        grid=(N_ENT // bm,),
        in_specs=[
            pl.BlockSpec((bm, CH), lambda m: (m, 0)),
            pl.BlockSpec((CH, CH), lambda m: (0, 0)),
        ],
        out_specs=pl.BlockSpec((bm, CH), lambda m: (m, 0)),
        out_shape=jax.ShapeDtypeStruct((N_ENT, CH), jnp.float32),
    )(x, w)


def _user_body(a_ref, b_ref, o_ref):
    acc = jnp.dot(a_ref[...], b_ref[...], preferred_element_type=jnp.float32)
    n = jnp.sqrt(jnp.sum(acc * acc, axis=1, keepdims=True))
    o_ref[...] = acc / jnp.maximum(n, 1e-12)


def _user_mm(interact, ent):
    bm = 512
    return pl.pallas_call(
        _user_body,
        grid=(N_USR // bm,),
        in_specs=[
            pl.BlockSpec((bm, N_ENT), lambda m: (m, 0)),
            pl.BlockSpec((N_ENT, CH), lambda m: (0, 0)),
        ],
        out_specs=pl.BlockSpec((bm, CH), lambda m: (m, 0)),
        out_shape=jax.ShapeDtypeStruct((N_USR, CH), jnp.float32),
    )(interact, ent)


def _combine_body(a_ref, b_ref, s_ref, o_ref):
    u = a_ref[0] + b_ref[0]
    s0 = s_ref[:, 0:1]
    s1 = s_ref[:, 1:2]
    lanes = lax.broadcasted_iota(jnp.int32, u.shape, 1)
    denom = jnp.where(lanes < DK, s0, s1) + 1e-16
    agg = u / denom
    n = jnp.sqrt(jnp.sum(agg * agg, axis=1, keepdims=True))
    o_ref[...] = agg / jnp.maximum(n, 1e-12)


def _combine(eo, s):
    bm = 2000
    return pl.pallas_call(
        _combine_body,
        grid=(N_ENT // bm,),
        in_specs=[
            pl.BlockSpec((1, bm, CH), lambda m: (0, m, 0)),
            pl.BlockSpec((1, bm, CH), lambda m: (1, m, 0)),
            pl.BlockSpec((bm, 2), lambda m: (m, 0)),
        ],
        out_specs=pl.BlockSpec((bm, CH), lambda m: (m, 0)),
        out_shape=jax.ShapeDtypeStruct((N_ENT, CH), jnp.float32),
    )(eo, eo, s)


_MESH = plsc.VectorSubcoreMesh(core_axis_name="c", subcore_axis_name="s")


@functools.partial(
    pl.kernel,
    out_type=pltpu.HBM((NC, ACC_ROWS, CH), jnp.float32),
    mesh=_MESH,
    compiler_params=pltpu.CompilerParams(needs_layout_passes=False),
    scratch_types=[
        pltpu.VMEM_SHARED((ACC_ROWS, CH), jnp.float32),
        pltpu.VMEM((4, B), jnp.int32),
        pltpu.VMEM((B, CH), jnp.int32),
        pltpu.VMEM((B, CH), jnp.int32),
        pltpu.VMEM((B, CH), jnp.float32),
        pltpu.VMEM((B, CH), jnp.float32),
        pltpu.VMEM((16, DK), jnp.int32),
        pltpu.SemaphoreType.DMA,
        pltpu.SemaphoreType.DMA,
        pltpu.SemaphoreType.DMA,
    ],
)
def _edge_sc(ttab, reltab, eidx, zinit, out,
             acc, hp, ph, tb, ov, erow, rloc, sm1, sm2, sm3):
    cid = lax.axis_index("c")
    sid = lax.axis_index("s")
    wid = sid * NC + cid

    lanes = lax.iota(jnp.int32, 16)
    m0 = lanes == 0
    mj = [lanes == j for j in range(16)]
    zv = jnp.zeros((16,), jnp.float32)

    @pl.when(sid == 0)
    def _():
        pltpu.sync_copy(zinit, acc)

    pltpu.sync_copy(reltab, rloc.at[pl.ds(0, N_REL11)])

    def zrow(r, carry):
        for c in range(8):
            erow[r, pl.ds(c * 16, 16)] = zv
        return carry

    lax.fori_loop(0, B, zrow, 0)
    plsc.subcore_barrier()

    def up2(v):
        return plsc.unpack(v, format=plsc.PackFormat.INTERLEAVED,
                           preferred_element_type=jnp.float32)

    def chunk(k, carry):
        pltpu.sync_copy(eidx.at[wid * NCHUNK + k], hp.at[pl.ds(0, 3)])
        c1 = pltpu.async_copy(ttab.at[hp.at[0]], ph, sm1)
        c2 = pltpu.async_copy(ttab.at[hp.at[1]], tb, sm2)
        c1.wait()
        c2.wait()

        def group(g, gcarry):
            h16 = hp[0, pl.ds(g * 16, 16)]
            r16 = hp[2, pl.ds(g * 16, 16)]
            iv = lanes + g * 16
            e0v = zv
            e1v = zv
            for j in range(16):
                i = g * 16 + j
                rj = r16[j]
                rb = [plsc.bitcast(rloc[rj, pl.ds(c * 16, 16)], jnp.bfloat16)
                      for c in range(4)]
                kf = []
                for c in range(4):
                    pb = plsc.bitcast(ph[i, pl.ds(c * 16, 16)], jnp.bfloat16)
                    tbw = plsc.bitcast(tb[i, pl.ds(c * 16, 16)], jnp.bfloat16)
                    kf.extend(up2((pb * tbw) * rb[c]))
                s0 = jnp.sum((kf[0] + kf[1]) + (kf[2] + kf[3]))
                s1 = jnp.sum((kf[4] + kf[5]) + (kf[6] + kf[7]))
                ev = jnp.exp(jnp.where(m0, s0, s1) * 0.125)
                e0 = ev[0]
                e1 = ev[1]
                for c in range(4):
                    vb = plsc.bitcast(tb[i, pl.ds(DK + c * 16, 16)],
                                      jnp.bfloat16)
                    lo, hi = up2(vb * rb[c])
                    e = e0 if c < 2 else e1
                    ov[i, pl.ds(c * 32, 16)] = lo * e
                    ov[i, pl.ds(c * 32 + 16, 16)] = hi * e
                e0v = jnp.where(mj[j], e0, e0v)
                e1v = jnp.where(mj[j], e1, e1v)
            colv = (h16 & 63) * 2
            hp[3, pl.ds(g * 16, 16)] = lax.shift_right_logical(h16, 6) + N_ENT
            plsc.store_scatter(erow, [iv, colv], e0v)
            plsc.store_scatter(erow, [iv, colv + 1], e1v)
            return gcarry

        lax.fori_loop(0, NG, group, 0)
        pltpu.sync_copy(ov, acc.at[hp.at[0]], add=True)
        pltpu.sync_copy(erow, acc.at[hp.at[3]], add=True)

        def gz(g, gcarry):
            h16 = hp[0, pl.ds(g * 16, 16)]
            iv = lanes + g * 16
            colv = (h16 & 63) * 2
            plsc.store_scatter(erow, [iv, colv], zv)
            plsc.store_scatter(erow, [iv, colv + 1], zv)
            return gcarry

        lax.fori_loop(0, NG, gz, 0)
        return carry

    lax.fori_loop(0, NCHUNK, chunk, 0)
    plsc.subcore_barrier()

    @pl.when(sid == 0)
    def _():
        pltpu.sync_copy(acc, out.at[cid])


def _ileave(x):
    n = x.shape[0]
    y = x.reshape(n, CH // 32, 2, 16).transpose(0, 1, 3, 2).reshape(n, CH)
    y16 = y.astype(jnp.bfloat16).reshape(n, CH // 2, 2)
    return lax.bitcast_convert_type(y16, jnp.int32)


def kernel(user_emb, entity_emb, edge_index, edge_type, interact_mat,
           relation_emb, W_Q):
    head = edge_index[0]
    tail = edge_index[1]
    ridx = (edge_type + (N_REL11 - 1)) % N_REL11
    eidx = (jnp.stack([head, tail, ridx])
            .reshape(3, NW, NCHUNK, B).transpose(1, 2, 0, 3)
            .reshape(NW * NCHUNK, 3, B))
    reltab = _ileave(relation_emb)
    zinit = jnp.zeros((ACC_ROWS, CH), jnp.float32)

    ent = entity_emb
    ent_res = entity_emb
    usr_res = user_emb
    for _ in range(2):
        p = _pq(ent, W_Q)
        ttab = jnp.concatenate([_ileave(p), _ileave(ent)], axis=1)
        eo = _edge_sc(ttab, reltab, eidx, zinit)
        s = (eo[0, N_ENT:] + eo[1, N_ENT:]).reshape(-1, 2)[:N_ENT]
        usr = _user_mm(interact_mat, ent)
        ent = _combine(eo, s)
        ent_res = ent_res + ent
        usr_res = usr_res + usr
    return ent_res, usr_res


# speedup vs baseline: 1.6425x; 1.0109x over previous
import functools

import jax
import jax.numpy as jnp
from jax import lax
from jax.experimental import pallas as pl
from jax.experimental.pallas import tpu as pltpu
from jax.experimental.pallas import tpu_sc as plsc

N_ENT = 10000
N_USR = 4096
N_EDGE = 320000
CH = 128
DK = 64
N_REL11 = 11

NC = 2
NS = 16
NW = NC * NS
EPW = N_EDGE // NW
B = 80
NG = B // 16
NCHUNK = EPW // B
CROWS = 160
ACC_ROWS = N_ENT + CROWS


def _pq_body(x_ref, w_ref, o_ref):
    o_ref[...] = jnp.dot(x_ref[...], w_ref[...],
                         preferred_element_type=jnp.float32)


def _pq(x, w):
    bm = 2000
    return pl.pallas_call(
        _pq_body,
        grid=(N_ENT // bm,),
        in_specs=[
            pl.BlockSpec((bm, CH), lambda m: (m, 0)),
            pl.BlockSpec((CH, CH), lambda m: (0, 0)),
        ],
        out_specs=pl.BlockSpec((bm, CH), lambda m: (m, 0)),
        out_shape=jax.ShapeDtypeStruct((N_ENT, CH), jnp.float32),
    )(x, w)


def _user_body(a_ref, b_ref, o_ref):
    acc = jnp.dot(a_ref[...], b_ref[...], preferred_element_type=jnp.float32)
    n = jnp.sqrt(jnp.sum(acc * acc, axis=1, keepdims=True))
    o_ref[...] = acc / jnp.maximum(n, 1e-12)


def _user_mm(interact, ent):
    bm = 512
    return pl.pallas_call(
        _user_body,
        grid=(N_USR // bm,),
        in_specs=[
            pl.BlockSpec((bm, N_ENT), lambda m: (m, 0)),
            pl.BlockSpec((N_ENT, CH), lambda m: (0, 0)),
        ],
        out_specs=pl.BlockSpec((bm, CH), lambda m: (m, 0)),
        out_shape=jax.ShapeDtypeStruct((N_USR, CH), jnp.float32),
    )(interact, ent)


def _combine_body(a_ref, b_ref, s_ref, o_ref):
    u = a_ref[0] + b_ref[0]
    s0 = s_ref[:, 0:1]
    s1 = s_ref[:, 1:2]
    lanes = lax.broadcasted_iota(jnp.int32, u.shape, 1)
    denom = jnp.where(lanes < DK, s0, s1) + 1e-16
    agg = u / denom
    n = jnp.sqrt(jnp.sum(agg * agg, axis=1, keepdims=True))
    o_ref[...] = agg / jnp.maximum(n, 1e-12)


def _combine(eo, s):
    bm = 2000
    return pl.pallas_call(
        _combine_body,
        grid=(N_ENT // bm,),
        in_specs=[
            pl.BlockSpec((1, bm, CH), lambda m: (0, m, 0)),
            pl.BlockSpec((1, bm, CH), lambda m: (1, m, 0)),
            pl.BlockSpec((bm, 2), lambda m: (m, 0)),
        ],
        out_specs=pl.BlockSpec((bm, CH), lambda m: (m, 0)),
        out_shape=jax.ShapeDtypeStruct((N_ENT, CH), jnp.float32),
    )(eo, eo, s)


_MESH = plsc.VectorSubcoreMesh(core_axis_name="c", subcore_axis_name="s")


@functools.partial(
    pl.kernel,
    out_type=pltpu.HBM((NC, ACC_ROWS, CH), jnp.float32),
    mesh=_MESH,
    compiler_params=pltpu.CompilerParams(needs_layout_passes=False),
    scratch_types=[
        pltpu.VMEM_SHARED((ACC_ROWS, CH), jnp.float32),
        pltpu.VMEM((4, B), jnp.int32),
        pltpu.VMEM((B, CH), jnp.int32),
        pltpu.VMEM((B, CH), jnp.int32),
        pltpu.VMEM((B, CH), jnp.float32),
        pltpu.VMEM((B, CH), jnp.float32),
        pltpu.VMEM((16, DK), jnp.int32),
        pltpu.SemaphoreType.DMA,
        pltpu.SemaphoreType.DMA,
        pltpu.SemaphoreType.DMA,
    ],
)
def _edge_sc(ttab, reltab, eidx, zinit, out,
             acc, hp, ph, tb, ov, erow, rloc, sm1, sm2, sm3):
    cid = lax.axis_index("c")
    sid = lax.axis_index("s")
    wid = sid * NC + cid

    lanes = lax.iota(jnp.int32, 16)
    m0 = lanes == 0
    mj = [lanes == j for j in range(16)]
    zv = jnp.zeros((16,), jnp.float32)

    @pl.when(sid == 0)
    def _():
        pltpu.sync_copy(zinit, acc)

    pltpu.sync_copy(reltab, rloc.at[pl.ds(0, N_REL11)])

    def zrow(r, carry):
        for c in range(8):
            erow[r, pl.ds(c * 16, 16)] = zv
        return carry

    lax.fori_loop(0, B, zrow, 0)
    plsc.subcore_barrier()

    def up2(v):
        return plsc.unpack(v, format=plsc.PackFormat.INTERLEAVED,
                           preferred_element_type=jnp.float32)

    def chunk(k, carry):
        pltpu.sync_copy(eidx.at[wid * NCHUNK + k], hp.at[pl.ds(0, 3)])
        c1 = pltpu.async_copy(ttab.at[hp.at[0]], ph, sm1)
        c2 = pltpu.async_copy(ttab.at[hp.at[1]], tb, sm2)
        c1.wait()
        c2.wait()

        def group(g, gcarry):
            h16 = hp[0, pl.ds(g * 16, 16)]
            r16 = hp[2, pl.ds(g * 16, 16)]
            iv = lanes + g * 16
            e0v = zv
            e1v = zv
            for j in range(16):
                i = g * 16 + j
                rj = r16[j]
                rb = [plsc.bitcast(rloc[rj, pl.ds(c * 16, 16)], jnp.bfloat16)
                      for c in range(4)]
                kf = []
                for c in range(4):
                    pb = plsc.bitcast(ph[i, pl.ds(c * 16, 16)], jnp.bfloat16)
                    tbw = plsc.bitcast(tb[i, pl.ds(c * 16, 16)], jnp.bfloat16)
                    kf.extend(up2((pb * tbw) * rb[c]))
                s0 = jnp.sum((kf[0] + kf[1]) + (kf[2] + kf[3]))
                s1 = jnp.sum((kf[4] + kf[5]) + (kf[6] + kf[7]))
                ev = jnp.exp(jnp.where(m0, s0, s1) * 0.125)
                e0 = ev[0]
                e1 = ev[1]
                for c in range(4):
                    vb = plsc.bitcast(tb[i, pl.ds(DK + c * 16, 16)],
                                      jnp.bfloat16)
                    lo, hi = up2(vb * rb[c])
                    e = e0 if c < 2 else e1
                    ov[i, pl.ds(c * 32, 16)] = lo * e
                    ov[i, pl.ds(c * 32 + 16, 16)] = hi * e
                e0v = jnp.where(mj[j], e0, e0v)
                e1v = jnp.where(mj[j], e1, e1v)
            colv = (h16 & 63) * 2
            hp[3, pl.ds(g * 16, 16)] = lax.shift_right_logical(h16, 6) + N_ENT
            plsc.store_scatter(erow, [iv, colv], e0v)
            plsc.store_scatter(erow, [iv, colv + 1], e1v)
            return gcarry

        lax.fori_loop(0, NG, group, 0)
        c3 = pltpu.async_copy(ov, acc.at[hp.at[0]], sm1, add=True)
        c4 = pltpu.async_copy(erow, acc.at[hp.at[3]], sm2, add=True)
        c3.wait()
        c4.wait()

        def gz(g, gcarry):
            h16 = hp[0, pl.ds(g * 16, 16)]
            iv = lanes + g * 16
            colv = (h16 & 63) * 2
            plsc.store_scatter(erow, [iv, colv], zv)
            plsc.store_scatter(erow, [iv, colv + 1], zv)
            return gcarry

        lax.fori_loop(0, NG, gz, 0)
        return carry

    lax.fori_loop(0, NCHUNK, chunk, 0)
    plsc.subcore_barrier()

    @pl.when(sid == 0)
    def _():
        pltpu.sync_copy(acc, out.at[cid])


def _ileave(x):
    n = x.shape[0]
    y = x.reshape(n, CH // 32, 2, 16).transpose(0, 1, 3, 2).reshape(n, CH)
    y16 = y.astype(jnp.bfloat16).reshape(n, CH // 2, 2)
    return lax.bitcast_convert_type(y16, jnp.int32)


def kernel(user_emb, entity_emb, edge_index, edge_type, interact_mat,
           relation_emb, W_Q):
    head = edge_index[0]
    tail = edge_index[1]
    ridx = (edge_type + (N_REL11 - 1)) % N_REL11
    eidx = (jnp.stack([head, tail, ridx])
            .reshape(3, NW, NCHUNK, B).transpose(1, 2, 0, 3)
            .reshape(NW * NCHUNK, 3, B))
    reltab = _ileave(relation_emb)
    zinit = jnp.zeros((ACC_ROWS, CH), jnp.float32)

    ent = entity_emb
    ent_res = entity_emb
    usr_res = user_emb
    for _ in range(2):
        p = _pq(ent, W_Q)
        ttab = jnp.concatenate([_ileave(p), _ileave(ent)], axis=1)
        eo = _edge_sc(ttab, reltab, eidx, zinit)
        s = (eo[0, N_ENT:] + eo[1, N_ENT:]).reshape(-1, 2)[:N_ENT]
        usr = _user_mm(interact_mat, ent)
        ent = _combine(eo, s)
        ent_res = ent_res + ent
        usr_res = usr_res + usr
    return ent_res, usr_res
